# Initial kernel scaffold; baseline (speedup 1.0000x reference)
import functools

import jax
import jax.numpy as jnp
from jax import lax
from jax.experimental import pallas as pl
from jax.experimental.pallas import tpu as pltpu
from jax.experimental.pallas import tpu_sc as plsc

N = 10000
E = 320000
NC, NS = 2, 16
NW = NC * NS
CHUNK = 80
EPW = E // NW
NCHUNK = EPW // CHUNK
RPS = N // NS
F32 = jnp.float32


def _mesh():
    return plsc.VectorSubcoreMesh(
        core_axis_name="c", subcore_axis_name="s",
        num_cores=NC, num_subcores=NS)


@functools.partial(
    pl.kernel,
    out_type=jax.ShapeDtypeStruct((NC, N), F32),
    mesh=_mesh(),
    scratch_types=[
        pltpu.VMEM((NCHUNK, CHUNK), jnp.int32),
        pltpu.VMEM((CHUNK,), F32),
        pltpu.VMEM_SHARED((N,), F32),
    ],
)
def _deg_kernel(col_hbm, zero_hbm, out_hbm, col_v, ones_v, accum):
    cid = lax.axis_index("c")
    sid = lax.axis_index("s")
    wid = cid * NS + sid
    for k in range(CHUNK // 16):
        ones_v[pl.ds(k * 16, 16)] = jnp.full((16,), 1.0, F32)
    pltpu.sync_copy(col_hbm.at[pl.ds(wid * NCHUNK, NCHUNK)], col_v)
    sl = pl.ds(sid * RPS, RPS)
    pltpu.sync_copy(zero_hbm.at[sl], accum.at[sl])
    plsc.subcore_barrier()

    def body(i, carry):
        pltpu.sync_copy(ones_v, accum.at[col_v.at[i]], add=True)
        return carry

    lax.fori_loop(0, NCHUNK, body, 0)
    plsc.subcore_barrier()
    pltpu.sync_copy(accum.at[sl], out_hbm.at[cid].at[sl])


def _make_agg(F):
    @functools.partial(
        pl.kernel,
        out_type=jax.ShapeDtypeStruct((NC, N, F), F32),
        mesh=_mesh(),
        scratch_types=[
            pltpu.VMEM((NCHUNK, CHUNK), jnp.int32),
            pltpu.VMEM((NCHUNK, CHUNK), jnp.int32),
            pltpu.VMEM((CHUNK, F), F32),
            pltpu.VMEM_SHARED((N, F), F32),
            pltpu.VMEM_SHARED((N, F), F32),
            pltpu.SemaphoreType.DMA,
        ],
    )
    def agg(g_hbm, row_hbm, col_hbm, out_hbm, row_v, col_v, rows_v,
            g_sp, accum, sem):
        cid = lax.axis_index("c")
        sid = lax.axis_index("s")
        wid = cid * NS + sid
        pltpu.sync_copy(row_hbm.at[pl.ds(wid * NCHUNK, NCHUNK)], row_v)
        pltpu.sync_copy(col_hbm.at[pl.ds(wid * NCHUNK, NCHUNK)], col_v)
        sl = pl.ds(sid * RPS, RPS)
        pltpu.sync_copy(g_hbm.at[sl], g_sp.at[sl])
        pltpu.sync_copy(g_hbm.at[sl], accum.at[sl])
        plsc.subcore_barrier()

        def body(i, carry):
            pltpu.async_copy(g_sp.at[row_v.at[i]], rows_v, sem).wait()
            pltpu.sync_copy(rows_v, accum.at[col_v.at[i]], add=True)
            return carry

        lax.fori_loop(0, NCHUNK, body, 0)
        plsc.subcore_barrier()
        pltpu.sync_copy(accum.at[sl], out_hbm.at[cid].at[sl])

    return agg


_agg32 = _make_agg(32)
_agg16 = _make_agg(16)


def _tc1_body(p0, p1, x, w1, dinv_o, g1_o):
    dinv = lax.rsqrt(p0[...] + p1[...] + 1.0)
    h = jnp.dot(x[...], w1[...], preferred_element_type=F32)
    dinv_o[...] = dinv
    g1_o[...] = h * dinv


_tc1 = pl.pallas_call(
    _tc1_body,
    out_shape=[jax.ShapeDtypeStruct((N, 1), F32),
               jax.ShapeDtypeStruct((N, 32), F32)],
)


def _mid_body(a0, a1, g, dinv, b, w, out):
    y = jnp.tanh((a0[...] + a1[...] - g[...]) * dinv[...] + b[...])
    out[...] = jnp.dot(y, w[...], preferred_element_type=F32) * dinv[...]


def _make_mid(f_out):
    return pl.pallas_call(
        _mid_body,
        out_shape=jax.ShapeDtypeStruct((N, f_out), F32),
    )


_mid_a = _make_mid(16)
_mid_b = _make_mid(16)


def _final_body(a0, a1, g, dinv, b3, wc, bc, out):
    y = jnp.tanh((a0[...] + a1[...] - g[...]) * dinv[...] + b3[...])
    out[...] = jnp.dot(y, wc[...], preferred_element_type=F32) + bc[...]


_tc_final = pl.pallas_call(
    _final_body,
    out_shape=jax.ShapeDtypeStruct((N, 4), F32),
)


@jax.jit
def kernel(x, edge_index, W1, b1, W2, b2, W3, b3, Wc, bc):
    ei = edge_index.astype(jnp.int32)
    row2 = ei[0].reshape(E // CHUNK, CHUNK)
    col2 = ei[1].reshape(E // CHUNK, CHUNK)
    zero = jnp.zeros((N,), F32)

    degp = _deg_kernel(col2, zero)
    p0 = degp[0].reshape(N, 1)
    p1 = degp[1].reshape(N, 1)

    dinv, g1 = _tc1(p0, p1, x, W1)

    a = _agg32(g1, row2, col2)
    g2 = _mid_a(a[0], a[1], g1, dinv, b1.reshape(1, 32), W2)

    a = _agg16(g2, row2, col2)
    w3p = jnp.pad(W3, ((0, 0), (0, 8)))
    g3 = _mid_b(a[0], a[1], g2, dinv, b2.reshape(1, 16), w3p)

    a = _agg16(g3, row2, col2)
    b3p = jnp.pad(b3, (0, 8)).reshape(1, 16)
    wcp = jnp.pad(Wc, ((0, 8), (0, 0)))
    return _tc_final(a[0], a[1], g3, dinv, b3p, wcp, bc.reshape(1, 4))


# speedup vs baseline: 23.8342x; 23.8342x over previous
import functools

import jax
import jax.numpy as jnp
from jax import lax
from jax.experimental import pallas as pl
from jax.experimental.pallas import tpu as pltpu
from jax.experimental.pallas import tpu_sc as plsc

N = 10000
E = 320000
NC, NS = 2, 16
NW = NC * NS
CHUNK = 80
EPW = E // NW
NCHUNK = EPW // CHUNK
RPS = 624
NPAD = 10240
F32 = jnp.float32


def _mesh():
    return plsc.VectorSubcoreMesh(
        core_axis_name="c", subcore_axis_name="s",
        num_cores=NC, num_subcores=NS)


@functools.partial(
    pl.kernel,
    out_type=jax.ShapeDtypeStruct((NC, NPAD), F32),
    mesh=_mesh(),
    scratch_types=[
        pltpu.VMEM((NCHUNK, CHUNK), jnp.int32),
        pltpu.VMEM((CHUNK,), F32),
        pltpu.VMEM_SHARED((NPAD,), F32),
    ],
)
def _deg_kernel(col_hbm, zero_hbm, out_hbm, col_v, ones_v, accum):
    cid = lax.axis_index("c")
    sid = lax.axis_index("s")
    wid = cid * NS + sid
    for k in range(CHUNK // 16):
        ones_v[pl.ds(k * 16, 16)] = jnp.full((16,), 1.0, F32)
    pltpu.sync_copy(col_hbm.at[wid], col_v)
    sl = pl.ds(sid * 640, 640)
    pltpu.sync_copy(zero_hbm.at[sl], accum.at[sl])
    plsc.subcore_barrier()

    def body(i, carry):
        pltpu.sync_copy(ones_v, accum.at[col_v.at[i]], add=True)
        return carry

    lax.fori_loop(0, NCHUNK, body, 0)
    plsc.subcore_barrier()
    pltpu.sync_copy(accum.at[sl], out_hbm.at[cid].at[sl])


_PROBE_LOOP = True


def _make_agg(F):
    @functools.partial(
        pl.kernel,
        out_type=jax.ShapeDtypeStruct((NC, NPAD, F), F32),
        mesh=_mesh(),
        scratch_types=[
            pltpu.VMEM((NCHUNK, CHUNK), jnp.int32),
            pltpu.VMEM((NCHUNK, CHUNK), jnp.int32),
            pltpu.VMEM((CHUNK, F), F32),
            pltpu.VMEM((NPAD // NS, F), F32),
            pltpu.VMEM_SHARED((NPAD, F), F32),
            pltpu.SemaphoreType.DMA,
        ],
        compiler_params=pltpu.CompilerParams(use_tc_tiling_on_sc=False),
    )
    def agg(g_hbm, row_hbm, col_hbm, out_hbm, row_v, col_v, rows_v,
            stage_v, accum, sem):
        cid = lax.axis_index("c")
        sid = lax.axis_index("s")
        wid = cid * NS + sid
        pltpu.sync_copy(row_hbm.at[wid], row_v)
        pltpu.sync_copy(col_hbm.at[wid], col_v)

        sl = pl.ds(sid * (NPAD // NS), NPAD // NS)
        pltpu.sync_copy(g_hbm.at[sl], stage_v)
        pltpu.sync_copy(stage_v, accum.at[sl])
        plsc.subcore_barrier()

        if _PROBE_LOOP:
            def body(i, carry):
                pltpu.async_copy(g_hbm.at[row_v.at[i]], rows_v, sem).wait()
                pltpu.sync_copy(rows_v, accum.at[col_v.at[i]], add=True)
                return carry

            lax.fori_loop(0, NCHUNK, body, 0)
            plsc.subcore_barrier()

        pltpu.sync_copy(accum.at[sl], stage_v)
        pltpu.sync_copy(stage_v, out_hbm.at[cid].at[sl])

    return agg


_agg32 = _make_agg(32)
_agg16 = _make_agg(16)


def _tc1_body(p0, p1, x, w1, dinv_o, g1_o):
    dinv = lax.rsqrt(p0[...] + p1[...] + 1.0)
    h = jnp.dot(x[...], w1[...], preferred_element_type=F32)
    dinv_o[...] = dinv
    g1_o[...] = h * dinv


_tc1 = pl.pallas_call(
    _tc1_body,
    out_shape=[jax.ShapeDtypeStruct((NPAD, 1), F32),
               jax.ShapeDtypeStruct((NPAD, 32), F32)],
)


def _mid_body(a0, a1, g, dinv, b, w, out):
    y = jnp.tanh((a0[...] + a1[...] - g[...]) * dinv[...] + b[...])
    out[...] = jnp.dot(y, w[...], preferred_element_type=F32) * dinv[...]


def _make_mid(f_out):
    return pl.pallas_call(
        _mid_body,
        out_shape=jax.ShapeDtypeStruct((NPAD, f_out), F32),
    )


_mid_a = _make_mid(16)
_mid_b = _make_mid(16)


def _final_body(a0, a1, g, dinv, b3, wc, bc, out):
    y = jnp.tanh((a0[...] + a1[...] - g[...]) * dinv[...] + b3[...])
    out[...] = jnp.dot(y, wc[...], preferred_element_type=F32) + bc[...]


_tc_final = pl.pallas_call(
    _final_body,
    out_shape=jax.ShapeDtypeStruct((NPAD, 4), F32),
)


@jax.jit
def kernel(x, edge_index, W1, b1, W2, b2, W3, b3, Wc, bc):
    ei = edge_index.astype(jnp.int32)
    row2 = ei[0].reshape(NW, NCHUNK, CHUNK)
    col2 = ei[1].reshape(NW, NCHUNK, CHUNK)
    zero = jnp.zeros((NPAD,), F32)

    degp = _deg_kernel(col2, zero)
    p0 = degp[0].reshape(NPAD, 1)
    p1 = degp[1].reshape(NPAD, 1)

    xp = jnp.pad(x, ((0, NPAD - N), (0, 0)))
    dinv, g1 = _tc1(p0, p1, xp, W1)

    a = _agg32(g1, row2, col2)
    g2 = _mid_a(a[0], a[1], g1, dinv, b1.reshape(1, 32), W2)

    a = _agg16(g2, row2, col2)
    w3p = jnp.pad(W3, ((0, 0), (0, 8)))
    g3 = _mid_b(a[0], a[1], g2, dinv, b2.reshape(1, 16), w3p)

    a = _agg16(g3, row2, col2)
    b3p = jnp.pad(b3, (0, 8)).reshape(1, 16)
    wcp = jnp.pad(Wc, ((0, 8), (0, 0)))
    out = _tc_final(a[0], a[1], g3, dinv, b3p, wcp, bc.reshape(1, 4))
    return out[:N]


# speedup vs baseline: 39.0701x; 1.6392x over previous
import functools

import jax
import jax.numpy as jnp
from jax import lax
from jax.experimental import pallas as pl
from jax.experimental.pallas import tpu as pltpu
from jax.experimental.pallas import tpu_sc as plsc

N = 10000
E = 320000
NC, NS = 2, 16
NW = NC * NS
CHUNK = 125
EPW = E // NW
NCHUNK = EPW // CHUNK
NPAD = 10240
F32 = jnp.float32


def _mesh():
    return plsc.VectorSubcoreMesh(
        core_axis_name="c", subcore_axis_name="s",
        num_cores=NC, num_subcores=NS)


@functools.partial(
    pl.kernel,
    out_type=jax.ShapeDtypeStruct((NC, NPAD), F32),
    mesh=_mesh(),
    scratch_types=[
        pltpu.VMEM((NCHUNK, CHUNK), jnp.int32),
        pltpu.VMEM((CHUNK,), F32),
        pltpu.VMEM_SHARED((NPAD,), F32),
    ],
)
def _deg_kernel(col_hbm, zero_hbm, out_hbm, col_v, ones_v, accum):
    cid = lax.axis_index("c")
    sid = lax.axis_index("s")
    wid = cid * NS + sid
    for k in range(0, CHUNK - 15, 16):
        ones_v[pl.ds(k, 16)] = jnp.full((16,), 1.0, F32)
    ones_v[pl.ds(CHUNK - 16, 16)] = jnp.full((16,), 1.0, F32)
    pltpu.sync_copy(col_hbm.at[wid], col_v)
    sl = pl.ds(sid * 640, 640)
    pltpu.sync_copy(zero_hbm.at[sl], accum.at[sl])
    plsc.subcore_barrier()

    def body(i, carry):
        pltpu.sync_copy(ones_v, accum.at[col_v.at[i]], add=True)
        return carry

    lax.fori_loop(0, NCHUNK, body, 0)
    plsc.subcore_barrier()
    pltpu.sync_copy(accum.at[sl], out_hbm.at[cid].at[sl])


def _make_agg(F):
    @functools.partial(
        pl.kernel,
        out_type=jax.ShapeDtypeStruct((NC, NPAD, F), F32),
        mesh=_mesh(),
        scratch_types=[
            pltpu.VMEM((NCHUNK, CHUNK), jnp.int32),
            pltpu.VMEM((NCHUNK, CHUNK), jnp.int32),
            pltpu.VMEM((CHUNK, F), F32),
            pltpu.VMEM((CHUNK, F), F32),
            pltpu.VMEM((NPAD // NS, F), F32),
            pltpu.VMEM_SHARED((NPAD, F), F32),
            pltpu.SemaphoreType.DMA,
            pltpu.SemaphoreType.DMA,
        ],
        compiler_params=pltpu.CompilerParams(use_tc_tiling_on_sc=False),
    )
    def agg(g_hbm, row_hbm, col_hbm, out_hbm, row_v, col_v, rows_a,
            rows_b, stage_v, accum, sem_a, sem_b):
        cid = lax.axis_index("c")
        sid = lax.axis_index("s")
        wid = cid * NS + sid
        pltpu.sync_copy(row_hbm.at[wid], row_v)
        pltpu.sync_copy(col_hbm.at[wid], col_v)

        sl = pl.ds(sid * (NPAD // NS), NPAD // NS)
        pltpu.sync_copy(g_hbm.at[sl], stage_v)
        pltpu.sync_copy(stage_v, accum.at[sl])
        plsc.subcore_barrier()

        def gather(j, buf, sem):
            return pltpu.async_copy(g_hbm.at[row_v.at[j]], buf, sem)

        def scatter(j, buf):
            pltpu.sync_copy(buf, accum.at[col_v.at[j]], add=True)

        gather(0, rows_a, sem_a)

        def body(k, carry):
            j = 2 * k
            gather(j + 1, rows_b, sem_b)
            pltpu.make_async_copy(g_hbm.at[row_v.at[j]], rows_a, sem_a).wait()
            scatter(j, rows_a)
            gather(j + 2, rows_a, sem_a)
            pltpu.make_async_copy(
                g_hbm.at[row_v.at[j]], rows_b, sem_b).wait()
            scatter(j + 1, rows_b)
            return carry

        lax.fori_loop(0, NCHUNK // 2 - 1, body, 0)
        gather(NCHUNK - 1, rows_b, sem_b)
        pltpu.make_async_copy(g_hbm.at[row_v.at[0]], rows_a, sem_a).wait()
        scatter(NCHUNK - 2, rows_a)
        pltpu.make_async_copy(g_hbm.at[row_v.at[0]], rows_b, sem_b).wait()
        scatter(NCHUNK - 1, rows_b)
        plsc.subcore_barrier()

        pltpu.sync_copy(accum.at[sl], stage_v)
        pltpu.sync_copy(stage_v, out_hbm.at[cid].at[sl])

    return agg


_agg32 = _make_agg(32)
_agg16 = _make_agg(16)


def _tc1_body(p0, p1, x, w1, dinv_o, g1_o):
    dinv = lax.rsqrt(p0[...] + p1[...] + 1.0)
    h = jnp.dot(x[...], w1[...], preferred_element_type=F32)
    dinv_o[...] = dinv
    g1_o[pl.ds(0, N), :] = h * dinv[:N, :]
    g1_o[pl.ds(N, NPAD - N), :] = jnp.zeros((NPAD - N, 32), F32)


_tc1 = pl.pallas_call(
    _tc1_body,
    out_shape=[jax.ShapeDtypeStruct((NPAD, 1), F32),
               jax.ShapeDtypeStruct((NPAD, 32), F32)],
)


def _mid_body(a0, a1, g, dinv, b, w, out):
    y = jnp.tanh((a0[...] + a1[...] - g[...]) * dinv[...] + b[...])
    out[...] = jnp.dot(y, w[...], preferred_element_type=F32) * dinv[...]


def _make_mid(f_out):
    return pl.pallas_call(
        _mid_body,
        out_shape=jax.ShapeDtypeStruct((NPAD, f_out), F32),
    )


_mid_a = _make_mid(16)
_mid_b = _make_mid(16)


def _final_body(a0, a1, g, dinv, b3, wc, bc, out):
    y = jnp.tanh((a0[...] + a1[...] - g[...]) * dinv[...] + b3[...])
    out[...] = jnp.dot(y, wc[...], preferred_element_type=F32) + bc[...]


_tc_final = pl.pallas_call(
    _final_body,
    out_shape=jax.ShapeDtypeStruct((NPAD, 4), F32),
)


@jax.jit
def kernel(x, edge_index, W1, b1, W2, b2, W3, b3, Wc, bc):
    ei = edge_index.astype(jnp.int32)
    row2 = ei[0].reshape(NW, NCHUNK, CHUNK)
    col2 = ei[1].reshape(NW, NCHUNK, CHUNK)
    zero = jnp.zeros((NPAD,), F32)

    degp = _deg_kernel(col2, zero)
    p0 = degp[0].reshape(NPAD, 1)
    p1 = degp[1].reshape(NPAD, 1)

    dinv, g1 = _tc1(p0, p1, x, W1)

    a = _agg32(g1, row2, col2)
    g2 = _mid_a(a[0], a[1], g1, dinv, b1.reshape(1, 32), W2)

    a = _agg16(g2, row2, col2)
    w3p = jnp.pad(W3, ((0, 0), (0, 8)))
    g3 = _mid_b(a[0], a[1], g2, dinv, b2.reshape(1, 16), w3p)

    a = _agg16(g3, row2, col2)
    b3p = jnp.pad(b3, (0, 8)).reshape(1, 16)
    wcp = jnp.pad(Wc, ((0, 8), (0, 0)))
    out = _tc_final(a[0], a[1], g3, dinv, b3p, wcp, bc.reshape(1, 4))
    return out[:N]


# speedup vs baseline: 45.5404x; 1.1656x over previous
import functools

import jax
import jax.numpy as jnp
from jax import lax
from jax.experimental import pallas as pl
from jax.experimental.pallas import tpu as pltpu
from jax.experimental.pallas import tpu_sc as plsc

N = 10000
E = 320000
NC, NS = 2, 16
NW = NC * NS
CHUNK = 125
EPW = E // NW
NCHUNK = EPW // CHUNK
NPAD = 10240
F32 = jnp.float32


def _mesh():
    return plsc.VectorSubcoreMesh(
        core_axis_name="c", subcore_axis_name="s",
        num_cores=NC, num_subcores=NS)


@functools.partial(
    pl.kernel,
    out_type=jax.ShapeDtypeStruct((NC, NPAD), F32),
    mesh=_mesh(),
    scratch_types=[
        pltpu.VMEM((NCHUNK, CHUNK), jnp.int32),
        pltpu.VMEM((CHUNK,), F32),
        pltpu.VMEM_SHARED((NPAD,), F32),
    ],
)
def _deg_kernel(col_hbm, zero_hbm, out_hbm, col_v, ones_v, accum):
    cid = lax.axis_index("c")
    sid = lax.axis_index("s")
    wid = cid * NS + sid
    for k in range(0, CHUNK - 15, 16):
        ones_v[pl.ds(k, 16)] = jnp.full((16,), 1.0, F32)
    ones_v[pl.ds(CHUNK - 16, 16)] = jnp.full((16,), 1.0, F32)
    pltpu.sync_copy(col_hbm.at[wid], col_v)
    sl = pl.ds(sid * 640, 640)
    pltpu.sync_copy(zero_hbm.at[sl], accum.at[sl])
    plsc.subcore_barrier()

    def body(i, carry):
        pltpu.sync_copy(ones_v, accum.at[col_v.at[i]], add=True)
        return carry

    lax.fori_loop(0, NCHUNK, body, 0)
    plsc.subcore_barrier()
    pltpu.sync_copy(accum.at[sl], out_hbm.at[cid].at[sl])


def _make_agg(F):
    @functools.partial(
        pl.kernel,
        out_type=jax.ShapeDtypeStruct((NC, NPAD, F), F32),
        mesh=_mesh(),
        scratch_types=[
            pltpu.VMEM((NCHUNK, CHUNK), jnp.int32),
            pltpu.VMEM((NCHUNK, CHUNK), jnp.int32),
            pltpu.VMEM((4, CHUNK, F), F32),
            pltpu.VMEM((NPAD // NS, F), F32),
            pltpu.VMEM_SHARED((NPAD, F), F32),
            [pltpu.SemaphoreType.DMA] * 4,
            [pltpu.SemaphoreType.DMA] * 4,
        ],
        compiler_params=pltpu.CompilerParams(use_tc_tiling_on_sc=False),
    )
    def agg(g_hbm, row_hbm, col_hbm, out_hbm, row_v, col_v, rows_v,
            stage_v, accum, gsem, ssem):
        cid = lax.axis_index("c")
        sid = lax.axis_index("s")
        wid = cid * NS + sid
        pltpu.sync_copy(row_hbm.at[wid], row_v)
        pltpu.sync_copy(col_hbm.at[wid], col_v)

        sl = pl.ds(sid * (NPAD // NS), NPAD // NS)
        pltpu.sync_copy(g_hbm.at[sl], stage_v)
        pltpu.sync_copy(stage_v, accum.at[sl])
        plsc.subcore_barrier()

        def gather(j, b):
            pltpu.async_copy(g_hbm.at[row_v.at[j]], rows_v.at[b], gsem[b])

        def wait_gather(b):
            pltpu.make_async_copy(
                g_hbm.at[row_v.at[0]], rows_v.at[b], gsem[b]).wait()

        def scatter(j, b):
            pltpu.async_copy(
                rows_v.at[b], accum.at[col_v.at[j]], ssem[b], add=True)

        def wait_scatter(j, b):
            pltpu.make_async_copy(
                rows_v.at[b], accum.at[col_v.at[j]], ssem[b]).wait()

        for b in range(4):
            gather(b, b)

        def body(k, carry):
            j0 = 4 * k
            for b in range(4):
                wait_gather(b)
                scatter(j0 + b, b)
            for b in range(4):
                wait_scatter(j0 + b, b)
                gather(j0 + 4 + b, b)
            return carry

        lax.fori_loop(0, NCHUNK // 4 - 1, body, 0)
        j0 = NCHUNK - 4
        for b in range(4):
            wait_gather(b)
            scatter(j0 + b, b)
        for b in range(4):
            wait_scatter(j0 + b, b)
        plsc.subcore_barrier()

        pltpu.sync_copy(accum.at[sl], stage_v)
        pltpu.sync_copy(stage_v, out_hbm.at[cid].at[sl])

    return agg


_agg32 = _make_agg(32)
_agg16 = _make_agg(16)


def _tc1_body(p0, p1, x, w1, dinv_o, g1_o):
    dinv = lax.rsqrt(p0[...] + p1[...] + 1.0)
    h = jnp.dot(x[...], w1[...], preferred_element_type=F32)
    dinv_o[...] = dinv
    g1_o[pl.ds(0, N), :] = h * dinv[:N, :]
    g1_o[pl.ds(N, NPAD - N), :] = jnp.zeros((NPAD - N, 32), F32)


_tc1 = pl.pallas_call(
    _tc1_body,
    out_shape=[jax.ShapeDtypeStruct((NPAD, 1), F32),
               jax.ShapeDtypeStruct((NPAD, 32), F32)],
)


def _mid_body(a0, a1, g, dinv, b, w, out):
    y = jnp.tanh((a0[...] + a1[...] - g[...]) * dinv[...] + b[...])
    out[...] = jnp.dot(y, w[...], preferred_element_type=F32) * dinv[...]


def _make_mid(f_out):
    return pl.pallas_call(
        _mid_body,
        out_shape=jax.ShapeDtypeStruct((NPAD, f_out), F32),
    )


_mid_a = _make_mid(16)
_mid_b = _make_mid(16)


def _final_body(a0, a1, g, dinv, b3, wc, bc, out):
    y = jnp.tanh((a0[...] + a1[...] - g[...]) * dinv[...] + b3[...])
    out[...] = jnp.dot(y, wc[...], preferred_element_type=F32) + bc[...]


_tc_final = pl.pallas_call(
    _final_body,
    out_shape=jax.ShapeDtypeStruct((NPAD, 4), F32),
)


@jax.jit
def kernel(x, edge_index, W1, b1, W2, b2, W3, b3, Wc, bc):
    ei = edge_index.astype(jnp.int32)
    row2 = ei[0].reshape(NW, NCHUNK, CHUNK)
    col2 = ei[1].reshape(NW, NCHUNK, CHUNK)
    zero = jnp.zeros((NPAD,), F32)

    degp = _deg_kernel(col2, zero)
    p0 = degp[0].reshape(NPAD, 1)
    p1 = degp[1].reshape(NPAD, 1)

    dinv, g1 = _tc1(p0, p1, x, W1)

    a = _agg32(g1, row2, col2)
    g2 = _mid_a(a[0], a[1], g1, dinv, b1.reshape(1, 32), W2)

    a = _agg16(g2, row2, col2)
    w3p = jnp.pad(W3, ((0, 0), (0, 8)))
    g3 = _mid_b(a[0], a[1], g2, dinv, b2.reshape(1, 16), w3p)

    a = _agg16(g3, row2, col2)
    b3p = jnp.pad(b3, (0, 8)).reshape(1, 16)
    wcp = jnp.pad(Wc, ((0, 8), (0, 0)))
    out = _tc_final(a[0], a[1], g3, dinv, b3p, wcp, bc.reshape(1, 4))
    return out[:N]


# speedup vs baseline: 48.8434x; 1.0725x over previous
import functools

import jax
import jax.numpy as jnp
from jax import lax
from jax.experimental import pallas as pl
from jax.experimental.pallas import tpu as pltpu
from jax.experimental.pallas import tpu_sc as plsc

N = 10000
E = 320000
NC, NS = 2, 16
NW = NC * NS
CHUNK = 125
EPW = E // NW
NCHUNK = EPW // CHUNK
NBUF = 8
NPAD = 10240
F32 = jnp.float32


def _mesh():
    return plsc.VectorSubcoreMesh(
        core_axis_name="c", subcore_axis_name="s",
        num_cores=NC, num_subcores=NS)


@functools.partial(
    pl.kernel,
    out_type=jax.ShapeDtypeStruct((NC, NPAD), F32),
    mesh=_mesh(),
    scratch_types=[
        pltpu.VMEM((NCHUNK, CHUNK), jnp.int32),
        pltpu.VMEM((CHUNK,), F32),
        pltpu.VMEM_SHARED((NPAD,), F32),
        pltpu.SemaphoreType.DMA,
    ],
)
def _deg_kernel(col_hbm, zero_hbm, out_hbm, col_v, ones_v, accum, sem):
    cid = lax.axis_index("c")
    sid = lax.axis_index("s")
    wid = cid * NS + sid
    for k in range(0, CHUNK - 15, 16):
        ones_v[pl.ds(k, 16)] = jnp.full((16,), 1.0, F32)
    ones_v[pl.ds(CHUNK - 16, 16)] = jnp.full((16,), 1.0, F32)
    pltpu.sync_copy(col_hbm.at[wid], col_v)
    sl = pl.ds(sid * 640, 640)
    pltpu.sync_copy(zero_hbm.at[sl], accum.at[sl])
    plsc.subcore_barrier()

    def body(i, carry):
        pltpu.async_copy(ones_v, accum.at[col_v.at[i]], sem, add=True)
        return carry

    lax.fori_loop(0, NCHUNK, body, 0)

    def drain(i, carry):
        pltpu.make_async_copy(ones_v, accum.at[col_v.at[0]], sem).wait()
        return carry

    lax.fori_loop(0, NCHUNK, drain, 0)
    plsc.subcore_barrier()
    pltpu.sync_copy(accum.at[sl], out_hbm.at[cid].at[sl])


def _make_agg(F):
    @functools.partial(
        pl.kernel,
        out_type=jax.ShapeDtypeStruct((NC, NPAD, F), F32),
        mesh=_mesh(),
        scratch_types=[
            pltpu.VMEM((NCHUNK, CHUNK), jnp.int32),
            pltpu.VMEM((NCHUNK, CHUNK), jnp.int32),
            pltpu.VMEM((NBUF, CHUNK, F), F32),
            pltpu.VMEM((NPAD // NS, F), F32),
            pltpu.VMEM_SHARED((NPAD, F), F32),
            [pltpu.SemaphoreType.DMA] * NBUF,
            [pltpu.SemaphoreType.DMA] * NBUF,
        ],
        compiler_params=pltpu.CompilerParams(use_tc_tiling_on_sc=False),
    )
    def agg(g_hbm, row_hbm, col_hbm, out_hbm, row_v, col_v, rows_v,
            stage_v, accum, gsem, ssem):
        cid = lax.axis_index("c")
        sid = lax.axis_index("s")
        wid = cid * NS + sid
        pltpu.sync_copy(row_hbm.at[wid], row_v)
        pltpu.sync_copy(col_hbm.at[wid], col_v)

        sl = pl.ds(sid * (NPAD // NS), NPAD // NS)
        pltpu.sync_copy(g_hbm.at[sl], stage_v)
        pltpu.sync_copy(stage_v, accum.at[sl])
        plsc.subcore_barrier()

        def gather(j, b):
            pltpu.async_copy(g_hbm.at[row_v.at[j]], rows_v.at[b], gsem[b])

        def wait_gather(b):
            pltpu.make_async_copy(
                g_hbm.at[row_v.at[0]], rows_v.at[b], gsem[b]).wait()

        def scatter(j, b):
            pltpu.async_copy(
                rows_v.at[b], accum.at[col_v.at[j]], ssem[b], add=True)

        def wait_scatter(j, b):
            pltpu.make_async_copy(
                rows_v.at[b], accum.at[col_v.at[j]], ssem[b]).wait()

        for b in range(NBUF):
            gather(b, b)

        def body(k, carry):
            j0 = NBUF * k
            for b in range(NBUF):
                wait_gather(b)
                scatter(j0 + b, b)
            for b in range(NBUF):
                wait_scatter(j0 + b, b)
                gather(j0 + NBUF + b, b)
            return carry

        lax.fori_loop(0, NCHUNK // NBUF - 1, body, 0)
        j0 = NCHUNK - NBUF
        for b in range(NBUF):
            wait_gather(b)
            scatter(j0 + b, b)
        for b in range(NBUF):
            wait_scatter(j0 + b, b)
        plsc.subcore_barrier()

        pltpu.sync_copy(accum.at[sl], stage_v)
        pltpu.sync_copy(stage_v, out_hbm.at[cid].at[sl])

    return agg


_agg32 = _make_agg(32)
_agg16 = _make_agg(16)


def _tc0_body(x, w1, h_o):
    h_o[...] = jnp.dot(x[...], w1[...], preferred_element_type=F32)


_tc0 = pl.pallas_call(
    _tc0_body,
    out_shape=jax.ShapeDtypeStruct((N, 32), F32),
)


def _tc1_body(p0, p1, h, dinv_o, g1_o):
    dinv = lax.rsqrt(p0[...] + p1[...] + 1.0)
    dinv_o[...] = dinv
    g1_o[pl.ds(0, N), :] = h[...] * dinv[:N, :]
    g1_o[pl.ds(N, NPAD - N), :] = jnp.zeros((NPAD - N, 32), F32)


_tc1 = pl.pallas_call(
    _tc1_body,
    out_shape=[jax.ShapeDtypeStruct((NPAD, 1), F32),
               jax.ShapeDtypeStruct((NPAD, 32), F32)],
)


def _mid_body(a0, a1, g, dinv, b, w, out):
    y = jnp.tanh((a0[...] + a1[...] - g[...]) * dinv[...] + b[...])
    out[...] = jnp.dot(y, w[...], preferred_element_type=F32) * dinv[...]


def _make_mid(f_out):
    return pl.pallas_call(
        _mid_body,
        out_shape=jax.ShapeDtypeStruct((NPAD, f_out), F32),
    )


_mid_a = _make_mid(16)
_mid_b = _make_mid(16)


def _final_body(a0, a1, g, dinv, b3, wc, bc, out):
    y = jnp.tanh((a0[...] + a1[...] - g[...]) * dinv[...] + b3[...])
    out[...] = jnp.dot(y, wc[...], preferred_element_type=F32) + bc[...]


_tc_final = pl.pallas_call(
    _final_body,
    out_shape=jax.ShapeDtypeStruct((NPAD, 4), F32),
)


@jax.jit
def kernel(x, edge_index, W1, b1, W2, b2, W3, b3, Wc, bc):
    ei = edge_index.astype(jnp.int32)
    row2 = ei[0].reshape(NW, NCHUNK, CHUNK)
    col2 = ei[1].reshape(NW, NCHUNK, CHUNK)
    zero = jnp.zeros((NPAD,), F32)

    degp = _deg_kernel(col2, zero)
    p0 = degp[0].reshape(NPAD, 1)
    p1 = degp[1].reshape(NPAD, 1)

    h1 = _tc0(x, W1)
    dinv, g1 = _tc1(p0, p1, h1)

    a = _agg32(g1, row2, col2)
    g2 = _mid_a(a[0], a[1], g1, dinv, b1.reshape(1, 32), W2)

    a = _agg16(g2, row2, col2)
    w3p = jnp.pad(W3, ((0, 0), (0, 8)))
    g3 = _mid_b(a[0], a[1], g2, dinv, b2.reshape(1, 16), w3p)

    a = _agg16(g3, row2, col2)
    b3p = jnp.pad(b3, (0, 8)).reshape(1, 16)
    wcp = jnp.pad(Wc, ((0, 8), (0, 0)))
    out = _tc_final(a[0], a[1], g3, dinv, b3p, wcp, bc.reshape(1, 4))
    return out[:N]
